# Initial kernel scaffold; baseline (speedup 1.0000x reference)
import jax
import jax.numpy as jnp
from jax.experimental import pallas as pl


def kernel(x, position, levels):
    raise NotImplementedError("write your pallas kernel here")


# speedup vs baseline: 1.5970x; 1.5970x over previous
import functools

import jax
import jax.numpy as jnp
from jax import lax
from jax.experimental import pallas as pl
from jax.experimental.pallas import tpu as pltpu
from jax.experimental.pallas import tpu_sc as plsc

B = 1024
SIZE = 26
D = 4096
NLEV = 100
LPAD = 104
TROWS = SIZE * LPAD

NC = 2
NS = 16
NW = NC * NS
B_PER_W = B // NW
LANES = 16


def _table_body(pos_ref, lev_ref, t_ref):
    p = pos_ref[0]
    lev = lev_ref[...]
    t_ref[...] = p + lev * (1.0 - 2.0 * p)


def _build_table(position, levels_pad):
    pos3 = position.reshape(SIZE, 1, D)
    return pl.pallas_call(
        _table_body,
        grid=(SIZE,),
        in_specs=[
            pl.BlockSpec((1, 1, D), lambda s: (s, 0, 0)),
            pl.BlockSpec((LPAD, D), lambda s: (0, 0)),
        ],
        out_specs=pl.BlockSpec((LPAD, D), lambda s: (s, 0)),
        out_shape=jax.ShapeDtypeStruct((TROWS, D), jnp.float32),
    )(pos3, levels_pad)


def _fidx_body(x_ref, out_ref):
    xv = x_ref[...]
    q = jnp.floor(xv * float(NLEV))
    q = jnp.clip(q, 0.0, float(NLEV - 1)).astype(jnp.int32)
    s = lax.broadcasted_iota(jnp.int32, (B, SIZE), 1)
    out_ref[...] = q + s * LPAD


def _build_fidx(x):
    return pl.pallas_call(
        _fidx_body,
        in_specs=[pl.BlockSpec((B, SIZE), lambda: (0, 0))],
        out_specs=pl.BlockSpec((B, SIZE), lambda: (0, 0)),
        out_shape=jax.ShapeDtypeStruct((B, SIZE), jnp.int32),
    )(x)


SL = 32
LN = D // SL


def _sc_gather_sum(table3, fidx):
    mesh = plsc.VectorSubcoreMesh(core_axis_name="c", subcore_axis_name="s")

    @functools.partial(
        pl.kernel,
        mesh=mesh,
        out_type=jax.ShapeDtypeStruct((B, SL, LN), jnp.float32),
        scratch_types=[
            pltpu.VMEM((B_PER_W, SIZE), jnp.int32),
            pltpu.VMEM((SIZE, SL, LN), jnp.float32),
            pltpu.VMEM((SL, LN), jnp.float32),
            pltpu.SemaphoreType.DMA,
        ],
    )
    def k(table_hbm, fidx_hbm, out_hbm, idx_v, rows_v, outrow_v, sem):
        wid = lax.axis_index("s") * NC + lax.axis_index("c")
        base = wid * B_PER_W
        pltpu.sync_copy(fidx_hbm.at[pl.ds(base, B_PER_W)], idx_v)

        def row_body(j, carry):
            pltpu.async_copy(table_hbm.at[idx_v.at[j]], rows_v, sem).wait()

            def chunk_body(c, carry2):
                sl = c // (LN // LANES)
                off = (c % (LN // LANES)) * LANES
                acc = rows_v[0, sl, pl.ds(off, LANES)]
                for s in range(1, SIZE):
                    acc = acc + rows_v[s, sl, pl.ds(off, LANES)]
                outrow_v[sl, pl.ds(off, LANES)] = acc
                return carry2

            lax.fori_loop(0, (SL * LN) // LANES, chunk_body, 0, unroll=False)
            pltpu.sync_copy(outrow_v, out_hbm.at[base + j])
            return carry

        lax.fori_loop(0, B_PER_W, row_body, 0, unroll=False)

    return k(table3, fidx)


def kernel(x, position, levels):
    levels_pad = jnp.pad(levels, ((0, LPAD - NLEV), (0, 0)))
    table = _build_table(position, levels_pad).reshape(TROWS, SL, LN)
    fidx = _build_fidx(x)
    out = _sc_gather_sum(table, fidx)
    return out.reshape(B, D)
